# baseline (speedup 1.0000x reference)
import jax
import jax.numpy as jnp
from jax.experimental import pallas as pl

B, N, S, D = 8, 512, 256, 64
DB = 32


def _tc_body(tt_ref, st_ref, out_ref):
    tt = tt_ref[...]
    st = st_ref[...]
    out_ref[0] = st[:, :, None] + tt[:, None, :]


def kernel(input_ids, time_table, space_table):
    del input_ids
    tt = time_table[:S].T
    st = space_table.T
    return pl.pallas_call(
        _tc_body,
        grid=(B, D // DB),
        in_specs=[
            pl.BlockSpec((DB, S), lambda b, j: (j, 0)),
            pl.BlockSpec((DB, N), lambda b, j: (j, 0)),
        ],
        out_specs=pl.BlockSpec((1, DB, N, S), lambda b, j: (b, j, 0, 0)),
        out_shape=jax.ShapeDtypeStruct((B, D, N, S), jnp.float32),
    )(tt, st)
